# baseline (speedup 1.0000x reference)
import functools

import jax
import jax.numpy as jnp
from jax import lax
from jax.experimental import pallas as pl
from jax.experimental.pallas import tpu as pltpu
from jax.experimental.pallas import tpu_sc as plsc

_ROWS = 1024

_NC = 2
_NS = 16
_NW = _NC * _NS


def _argmin_body(x_ref, insq_ref, cb_ref, cbsq_ref, idx_ref):
    x = x_ref[...]
    cb = cb_ref[...]
    mm = lax.dot_general(cb, x, (((1,), (1,)), ((), ())),
                         preferred_element_type=jnp.float32)
    d = (cbsq_ref[...] + insq_ref[...]) - 2.0 * mm
    k = d.shape[0]
    min_d = jnp.min(d, axis=0, keepdims=True)
    iota = lax.broadcasted_iota(jnp.int32, d.shape, 0)
    idx = jnp.min(jnp.where(d == min_d, iota, jnp.int32(k)), axis=0)
    idx_ref[...] = idx


def _argmin_call(x, insq, codebook, cbsq, row0, nrows):
    d_ = x.shape[1]
    k = codebook.shape[0]
    r0b = row0 // _ROWS
    return pl.pallas_call(
        _argmin_body,
        grid=(nrows // _ROWS,),
        in_specs=[
            pl.BlockSpec((_ROWS, d_), lambda i: (i + r0b, 0)),
            pl.BlockSpec((1, _ROWS), lambda i: (0, i + r0b)),
            pl.BlockSpec((k, d_), lambda i: (0, 0)),
            pl.BlockSpec((k, 1), lambda i: (0, 0)),
        ],
        out_specs=pl.BlockSpec((_ROWS,), lambda i: (i,)),
        out_shape=jax.ShapeDtypeStruct((nrows,), jnp.int32),
    )(x, insq, codebook, cbsq)


@functools.lru_cache(maxsize=None)
def _make_gather(nb, t_, d_, k):
    b_per_w = nb // _NW
    n_per_w = b_per_w * t_
    groups = t_ // 16
    mesh = plsc.VectorSubcoreMesh(core_axis_name="c", subcore_axis_name="s")

    @functools.partial(
        pl.kernel, mesh=mesh,
        compiler_params=pltpu.CompilerParams(use_tc_tiling_on_sc=False,
                                             needs_layout_passes=False),
        out_type=jax.ShapeDtypeStruct((nb, d_ * t_), jnp.float32),
        scratch_types=[
            pltpu.VMEM((d_ * k,), jnp.float32),
            pltpu.VMEM((n_per_w,), jnp.int32),
            pltpu.VMEM((d_ * t_,), jnp.float32),
        ],
    )
    def gk(cbt_hbm, idx_hbm, out_hbm, cbt_v, idx_v, zqt_v):
        wid = lax.axis_index("s") * _NC + lax.axis_index("c")
        pltpu.sync_copy(cbt_hbm, cbt_v)
        pltpu.sync_copy(idx_hbm.at[pl.ds(wid * n_per_w, n_per_w)], idx_v)

        for bb in range(b_per_w):
            @pl.loop(0, groups, unroll=2)
            def per_group(g, bb=bb):
                iv = idx_v[pl.ds(bb * t_ + g * 16, 16)]
                toff = g * 16
                for dd in range(d_):
                    vals = plsc.load_gather(cbt_v, [iv + jnp.int32(dd * k)])
                    zqt_v[pl.ds(dd * t_ + toff, 16)] = vals

            b = wid * b_per_w + bb
            pltpu.sync_copy(zqt_v, out_hbm.at[b])

    return gk


def kernel(z_e_x, codebook):
    nb, t_, d_ = z_e_x.shape
    k = codebook.shape[0]
    x = z_e_x.reshape(-1, d_)
    n = x.shape[0]
    insq = jnp.sum(x ** 2, axis=1)[None, :]
    cbsq = jnp.sum(codebook ** 2, axis=1)[:, None]
    cbt = codebook.T.reshape(-1)
    half_n, half_b = n // 2, nb // 2
    gather = _make_gather(half_b, t_, d_, k)
    idx0 = _argmin_call(x, insq, codebook, cbsq, 0, half_n)
    idx1 = _argmin_call(x, insq, codebook, cbsq, half_n, half_n)
    zqt0 = gather(cbt, idx0)
    zqt1 = gather(cbt, idx1)
    zqt = jnp.concatenate([zqt0, zqt1], axis=0).reshape(nb, d_, t_)
    indices = jnp.concatenate([idx0, idx1])
    z_q = jnp.transpose(zqt, (0, 2, 1))
    return (z_q, z_q, indices)


# speedup vs baseline: 1.0557x; 1.0557x over previous
import functools

import jax
import jax.numpy as jnp
from jax import lax
from jax.experimental import pallas as pl
from jax.experimental.pallas import tpu as pltpu
from jax.experimental.pallas import tpu_sc as plsc

_ROWS = 1024

_NC = 2
_NS = 16
_NW = _NC * _NS


def _argmin_body(x_ref, insq_ref, cb_ref, cbsq_ref, idx_ref):
    x = x_ref[...]
    cb = cb_ref[...]
    mm = lax.dot_general(cb, x, (((1,), (1,)), ((), ())),
                         preferred_element_type=jnp.float32)
    d = (cbsq_ref[...] + insq_ref[...]) - 2.0 * mm
    k = d.shape[0]
    min_d = jnp.min(d, axis=0, keepdims=True)
    iota = lax.broadcasted_iota(jnp.int32, d.shape, 0)
    idx = jnp.min(jnp.where(d == min_d, iota, jnp.int32(k)), axis=0)
    idx_ref[...] = idx


def _argmin_call(x, insq, codebook, cbsq, row0, nrows):
    d_ = x.shape[1]
    k = codebook.shape[0]
    r0b = row0 // _ROWS
    return pl.pallas_call(
        _argmin_body,
        grid=(nrows // _ROWS,),
        in_specs=[
            pl.BlockSpec((_ROWS, d_), lambda i: (i + r0b, 0)),
            pl.BlockSpec((1, _ROWS), lambda i: (0, i + r0b)),
            pl.BlockSpec((k, d_), lambda i: (0, 0)),
            pl.BlockSpec((k, 1), lambda i: (0, 0)),
        ],
        out_specs=pl.BlockSpec((_ROWS,), lambda i: (i,)),
        out_shape=jax.ShapeDtypeStruct((nrows,), jnp.int32),
    )(x, insq, codebook, cbsq)


@functools.lru_cache(maxsize=None)
def _make_gather(nb, t_, d_, k):
    b_per_w = nb // _NW
    n_per_w = b_per_w * t_
    groups = t_ // 16
    mesh = plsc.VectorSubcoreMesh(core_axis_name="c", subcore_axis_name="s")

    @functools.partial(
        pl.kernel, mesh=mesh,
        compiler_params=pltpu.CompilerParams(use_tc_tiling_on_sc=False,
                                             needs_layout_passes=False),
        out_type=jax.ShapeDtypeStruct((nb, d_ * t_), jnp.float32),
        scratch_types=[
            pltpu.VMEM((d_ * k,), jnp.float32),
            pltpu.VMEM((n_per_w,), jnp.int32),
            pltpu.VMEM((d_ * t_,), jnp.float32),
        ],
    )
    def gk(cbt_hbm, idx_hbm, out_hbm, cbt_v, idx_v, zqt_v):
        wid = lax.axis_index("s") * _NC + lax.axis_index("c")
        pltpu.sync_copy(cbt_hbm, cbt_v)
        pltpu.sync_copy(idx_hbm.at[pl.ds(wid * n_per_w, n_per_w)], idx_v)

        for bb in range(b_per_w):
            @plsc.parallel_loop(0, groups)
            def per_group(g, bb=bb):
                iv = idx_v[pl.ds(bb * t_ + g * 16, 16)]
                toff = g * 16
                for dd in range(d_):
                    vals = plsc.load_gather(cbt_v, [iv + jnp.int32(dd * k)])
                    zqt_v[pl.ds(dd * t_ + toff, 16)] = vals

            b = wid * b_per_w + bb
            pltpu.sync_copy(zqt_v, out_hbm.at[b])

    return gk


def kernel(z_e_x, codebook):
    nb, t_, d_ = z_e_x.shape
    k = codebook.shape[0]
    x = z_e_x.reshape(-1, d_)
    n = x.shape[0]
    insq = jnp.sum(x ** 2, axis=1)[None, :]
    cbsq = jnp.sum(codebook ** 2, axis=1)[:, None]
    cbt = codebook.T.reshape(-1)
    half_n, half_b = n // 2, nb // 2
    gather = _make_gather(half_b, t_, d_, k)
    idx0 = _argmin_call(x, insq, codebook, cbsq, 0, half_n)
    idx1 = _argmin_call(x, insq, codebook, cbsq, half_n, half_n)
    zqt0 = gather(cbt, idx0)
    zqt1 = gather(cbt, idx1)
    zqt = jnp.concatenate([zqt0, zqt1], axis=0).reshape(nb, d_, t_)
    indices = jnp.concatenate([idx0, idx1])
    z_q = jnp.transpose(zqt, (0, 2, 1))
    return (z_q, z_q, indices)


# speedup vs baseline: 1.1329x; 1.0731x over previous
import functools

import jax
import jax.numpy as jnp
from jax import lax
from jax.experimental import pallas as pl
from jax.experimental.pallas import tpu as pltpu
from jax.experimental.pallas import tpu_sc as plsc

_ROWS = 1024

_NC = 2
_NS = 16
_NW = _NC * _NS


def _argmin_body(x_ref, insq_ref, cb_ref, cbsq_ref, idx_ref):
    x = x_ref[...]
    cb = cb_ref[...]
    mm = lax.dot_general(cb, x, (((1,), (1,)), ((), ())),
                         preferred_element_type=jnp.float32)
    d = (cbsq_ref[...] + insq_ref[...]) - 2.0 * mm
    k = d.shape[0]
    min_d = jnp.min(d, axis=0, keepdims=True)
    iota = lax.broadcasted_iota(jnp.int32, d.shape, 0)
    idx = jnp.min(jnp.where(d == min_d, iota, jnp.int32(k)), axis=0)
    idx_ref[...] = idx


def _argmin_call(x, insq, codebook, cbsq, row0, nrows):
    d_ = x.shape[1]
    k = codebook.shape[0]
    r0b = row0 // _ROWS
    return pl.pallas_call(
        _argmin_body,
        grid=(nrows // _ROWS,),
        in_specs=[
            pl.BlockSpec((_ROWS, d_), lambda i: (i + r0b, 0)),
            pl.BlockSpec((1, _ROWS), lambda i: (0, i + r0b)),
            pl.BlockSpec((k, d_), lambda i: (0, 0)),
            pl.BlockSpec((k, 1), lambda i: (0, 0)),
        ],
        out_specs=pl.BlockSpec((_ROWS,), lambda i: (i,)),
        out_shape=jax.ShapeDtypeStruct((nrows,), jnp.int32),
    )(x, insq, codebook, cbsq)


@functools.lru_cache(maxsize=None)
def _make_gather(nb, t_, d_, k):
    b_per_w = nb // _NW
    n_per_w = b_per_w * t_
    groups = t_ // 16
    mesh = plsc.VectorSubcoreMesh(core_axis_name="c", subcore_axis_name="s")

    @functools.partial(
        pl.kernel, mesh=mesh,
        compiler_params=pltpu.CompilerParams(use_tc_tiling_on_sc=False,
                                             needs_layout_passes=False),
        out_type=jax.ShapeDtypeStruct((nb, d_, t_), jnp.float32),
        scratch_types=[
            pltpu.VMEM((d_, k), jnp.float32),
            pltpu.VMEM((n_per_w,), jnp.int32),
            pltpu.VMEM((d_, t_), jnp.float32),
        ],
    )
    def gk(cbt_hbm, idx_hbm, out_hbm, cbt_v, idx_v, zqt_v):
        wid = lax.axis_index("s") * _NC + lax.axis_index("c")
        pltpu.sync_copy(cbt_hbm, cbt_v)
        pltpu.sync_copy(idx_hbm.at[pl.ds(wid * n_per_w, n_per_w)], idx_v)

        for bb in range(b_per_w):
            @plsc.parallel_loop(0, groups)
            def per_group(g, bb=bb):
                iv = idx_v[pl.ds(bb * t_ + g * 16, 16)]
                for dd in range(d_):
                    row_sel = jnp.full((16,), dd, dtype=jnp.int32)
                    vals = plsc.load_gather(cbt_v, [row_sel, iv])
                    zqt_v[dd, pl.ds(g * 16, 16)] = vals

            b = wid * b_per_w + bb
            pltpu.sync_copy(zqt_v, out_hbm.at[b])

    return gk


def kernel(z_e_x, codebook):
    nb, t_, d_ = z_e_x.shape
    k = codebook.shape[0]
    x = z_e_x.reshape(-1, d_)
    n = x.shape[0]
    insq = jnp.sum(x ** 2, axis=1)[None, :]
    cbsq = jnp.sum(codebook ** 2, axis=1)[:, None]
    cbt = codebook.T
    half_n, half_b = n // 2, nb // 2
    gather = _make_gather(half_b, t_, d_, k)
    idx0 = _argmin_call(x, insq, codebook, cbsq, 0, half_n)
    idx1 = _argmin_call(x, insq, codebook, cbsq, half_n, half_n)
    zqt0 = gather(cbt, idx0)
    zqt1 = gather(cbt, idx1)
    zqt = jnp.concatenate([zqt0, zqt1], axis=0)
    indices = jnp.concatenate([idx0, idx1])
    z_q = jnp.transpose(zqt, (0, 2, 1))
    return (z_q, z_q, indices)


# speedup vs baseline: 1.2671x; 1.1185x over previous
import functools

import jax
import jax.numpy as jnp
from jax import lax
from jax.experimental import pallas as pl
from jax.experimental.pallas import tpu as pltpu
from jax.experimental.pallas import tpu_sc as plsc

_ROWS = 1024

_NC = 2
_NS = 16
_NW = _NC * _NS


def _argmin_body(x_ref, insq_ref, cb_ref, cbsq_ref, idx_ref):
    x = x_ref[...]
    cb = cb_ref[...]
    mm = lax.dot_general(cb, x, (((1,), (1,)), ((), ())),
                         preferred_element_type=jnp.float32)
    d = (cbsq_ref[...] + insq_ref[...]) - 2.0 * mm
    idx_ref[...] = jnp.argmin(d, axis=0).astype(jnp.int32)


def _argmin_call(x, insq, codebook, cbsq, row0, nrows):
    d_ = x.shape[1]
    k = codebook.shape[0]
    r0b = row0 // _ROWS
    return pl.pallas_call(
        _argmin_body,
        grid=(nrows // _ROWS,),
        in_specs=[
            pl.BlockSpec((_ROWS, d_), lambda i: (i + r0b, 0)),
            pl.BlockSpec((1, _ROWS), lambda i: (0, i + r0b)),
            pl.BlockSpec((k, d_), lambda i: (0, 0)),
            pl.BlockSpec((k, 1), lambda i: (0, 0)),
        ],
        out_specs=pl.BlockSpec((_ROWS,), lambda i: (i,)),
        out_shape=jax.ShapeDtypeStruct((nrows,), jnp.int32),
    )(x, insq, codebook, cbsq)


@functools.lru_cache(maxsize=None)
def _make_gather(nb, t_, d_, k):
    b_per_w = nb // _NW
    n_per_w = b_per_w * t_
    groups = t_ // 16
    mesh = plsc.VectorSubcoreMesh(core_axis_name="c", subcore_axis_name="s")

    @functools.partial(
        pl.kernel, mesh=mesh,
        compiler_params=pltpu.CompilerParams(use_tc_tiling_on_sc=False,
                                             needs_layout_passes=False),
        out_type=jax.ShapeDtypeStruct((nb, d_, t_), jnp.float32),
        scratch_types=[
            pltpu.VMEM((d_, k), jnp.float32),
            pltpu.VMEM((n_per_w,), jnp.int32),
            pltpu.VMEM((d_, t_), jnp.float32),
        ],
    )
    def gk(cbt_hbm, idx_hbm, out_hbm, cbt_v, idx_v, zqt_v):
        wid = lax.axis_index("s") * _NC + lax.axis_index("c")
        pltpu.sync_copy(cbt_hbm, cbt_v)
        pltpu.sync_copy(idx_hbm.at[pl.ds(wid * n_per_w, n_per_w)], idx_v)

        for bb in range(b_per_w):
            @plsc.parallel_loop(0, groups)
            def per_group(g, bb=bb):
                iv = idx_v[pl.ds(bb * t_ + g * 16, 16)]
                for dd in range(d_):
                    row_sel = jnp.full((16,), dd, dtype=jnp.int32)
                    vals = plsc.load_gather(cbt_v, [row_sel, iv])
                    zqt_v[dd, pl.ds(g * 16, 16)] = vals

            b = wid * b_per_w + bb
            pltpu.sync_copy(zqt_v, out_hbm.at[b])

    return gk


def kernel(z_e_x, codebook):
    nb, t_, d_ = z_e_x.shape
    k = codebook.shape[0]
    x = z_e_x.reshape(-1, d_)
    n = x.shape[0]
    insq = jnp.sum(x ** 2, axis=1)[None, :]
    cbsq = jnp.sum(codebook ** 2, axis=1)[:, None]
    cbt = codebook.T
    half_n, half_b = n // 2, nb // 2
    gather = _make_gather(half_b, t_, d_, k)
    idx0 = _argmin_call(x, insq, codebook, cbsq, 0, half_n)
    idx1 = _argmin_call(x, insq, codebook, cbsq, half_n, half_n)
    zqt0 = gather(cbt, idx0)
    zqt1 = gather(cbt, idx1)
    zqt = jnp.concatenate([zqt0, zqt1], axis=0)
    indices = jnp.concatenate([idx0, idx1])
    z_q = jnp.transpose(zqt, (0, 2, 1))
    return (z_q, z_q, indices)
